# Initial kernel scaffold; baseline (speedup 1.0000x reference)
import jax
import jax.numpy as jnp
from jax.experimental import pallas as pl


def kernel(emb_entity, emb_mole, entity_mole_weights, W1, root1, b1, W2, root2, b2):
    raise NotImplementedError("write your pallas kernel here")


# speedup vs baseline: 1113.6418x; 1113.6418x over previous
import functools

import jax
import jax.numpy as jnp
from jax.experimental import pallas as pl
from jax.experimental.pallas import tpu as pltpu

_LEVELS = 6


def _both_layers_kernel(wt_ref, x_ref, W1_ref, r1_ref, b1_ref, W2_ref,
                        r2_ref, b2_ref, out_ref):
    wt = wt_ref[...]
    n_med = wt.shape[0]

    masks = []
    cnts = []
    for r in range(1, _LEVELS):
        m = ((wt > r / _LEVELS) & (wt <= (r + 1) / _LEVELS)).astype(jnp.float32)
        masks.append(m)
        cnts.append(jnp.maximum(jnp.sum(m, axis=1, keepdims=True), 1.0))

    def layer(x, W_ref, root_ref, b_ref):
        agg = jnp.zeros((n_med, x.shape[1]), dtype=jnp.float32)
        for k in range(_LEVELS - 1):
            g = jnp.dot(masks[k], x, preferred_element_type=jnp.float32)
            agg = agg + jnp.dot(g / cnts[k], W_ref[k],
                                preferred_element_type=jnp.float32)
        rt = jnp.dot(x, root_ref[...],
                     preferred_element_type=jnp.float32) + b_ref[...]
        top = rt[:n_med, :] + agg
        return jnp.concatenate([top, rt[n_med:, :]], axis=0)

    h1 = jnp.maximum(layer(x_ref[...], W1_ref, r1_ref, b1_ref), 0.0)
    out_ref[...] = layer(h1, W2_ref, r2_ref, b2_ref)


@jax.jit
def kernel(emb_entity, emb_mole, entity_mole_weights, W1, root1, b1, W2,
           root2, b2):
    del emb_mole
    x = emb_entity[0]
    n_ent, d = x.shape
    wt = entity_mole_weights.T

    out = pl.pallas_call(
        _both_layers_kernel,
        out_shape=jax.ShapeDtypeStruct((n_ent, d), jnp.float32),
    )(wt, x, W1[1:], root1, b1.reshape(1, d), W2[1:], root2,
      b2.reshape(1, d))
    return out


# speedup vs baseline: 1827.3910x; 1.6409x over previous
import jax
import jax.numpy as jnp
from jax.experimental import pallas as pl
from jax.experimental.pallas import tpu as pltpu

_LEVELS = 6


def _fused_kernel(w_ref, x_ref, W1_ref, r1_ref, b1_ref, W2_ref, r2_ref,
                  b2_ref, out_ref):
    wt = w_ref[...].T
    n_med = wt.shape[0]

    masks = []
    inv_cnts = []
    for r in range(1, _LEVELS):
        m = ((wt > r / _LEVELS) & (wt <= (r + 1) / _LEVELS)).astype(jnp.float32)
        inv_cnts.append(1.0 / jnp.maximum(
            jnp.sum(m, axis=1, keepdims=True), 1.0))
        masks.append(m.astype(jnp.bfloat16))

    def layer(x, W_ref, root_ref, b_ref):
        xb = x.astype(jnp.bfloat16)
        agg = jnp.zeros((n_med, x.shape[1]), dtype=jnp.float32)
        for k in range(_LEVELS - 1):
            g = jnp.dot(masks[k], xb, preferred_element_type=jnp.float32)
            agg = agg + jnp.dot(g * inv_cnts[k], W_ref[k + 1],
                                preferred_element_type=jnp.float32)
        rt = jnp.dot(xb, root_ref[...].astype(jnp.bfloat16),
                     preferred_element_type=jnp.float32) + b_ref[...]
        top = rt[:n_med, :] + agg
        return jnp.concatenate([top, rt[n_med:, :]], axis=0)

    h1 = jnp.maximum(layer(x_ref[...], W1_ref, r1_ref, b1_ref), 0.0)
    out_ref[...] = layer(h1, W2_ref, r2_ref, b2_ref)


@jax.jit
def kernel(emb_entity, emb_mole, entity_mole_weights, W1, root1, b1, W2,
           root2, b2):
    del emb_mole
    n_ent, d = emb_entity.shape[1], emb_entity.shape[2]
    x = emb_entity.reshape(n_ent, d)

    out = pl.pallas_call(
        _fused_kernel,
        out_shape=jax.ShapeDtypeStruct((n_ent, d), jnp.float32),
    )(entity_mole_weights, x, W1, root1, b1.reshape(1, d), W2, root2,
      b2.reshape(1, d))
    return out
